# baseline (speedup 1.0000x reference)
import functools

import jax
import jax.numpy as jnp
from jax import lax
from jax.experimental import pallas as pl
from jax.experimental.pallas import tpu as pltpu
from jax.experimental.pallas import tpu_sc as plsc

NUM_CLASSES = 100000
EMB_DIM = 128
BATCH = 16384

_NC = 2
_NS = 16
_NW = _NC * _NS

_C = 2
_BC = BATCH // _C
_BPW = _BC // _NW
_SUB = _BPW // 2


def _make_sc_gather(chunk_off):
    mesh = plsc.VectorSubcoreMesh(core_axis_name="c", subcore_axis_name="s")

    @functools.partial(
        pl.kernel,
        mesh=mesh,
        out_type=jax.ShapeDtypeStruct((_BC, EMB_DIM), jnp.float32),
        scratch_types=[
            pltpu.VMEM((_SUB,), jnp.int32),
            pltpu.VMEM((_SUB,), jnp.int32),
            pltpu.VMEM((_SUB, EMB_DIM), jnp.float32),
            pltpu.VMEM((_SUB, EMB_DIM), jnp.float32),
            pltpu.SemaphoreType.DMA,
            pltpu.SemaphoreType.DMA,
            pltpu.SemaphoreType.DMA,
            pltpu.SemaphoreType.DMA,
        ],
    )
    def gather_k(labels_hbm, table_hbm, out_hbm,
                 idx0_v, idx1_v, rows0_v, rows1_v, g0, g1, w0, w1):
        wid = lax.axis_index("s") * _NC + lax.axis_index("c")
        base = wid * _BPW
        pltpu.sync_copy(labels_hbm.at[pl.ds(chunk_off + base, _SUB)], idx0_v)
        cp0 = pltpu.async_copy(table_hbm.at[idx0_v], rows0_v, g0)
        pltpu.sync_copy(
            labels_hbm.at[pl.ds(chunk_off + base + _SUB, _SUB)], idx1_v
        )
        cp1 = pltpu.async_copy(table_hbm.at[idx1_v], rows1_v, g1)
        cp0.wait()
        wb0 = pltpu.async_copy(rows0_v, out_hbm.at[pl.ds(base, _SUB)], w0)
        cp1.wait()
        wb1 = pltpu.async_copy(
            rows1_v, out_hbm.at[pl.ds(base + _SUB, _SUB)], w1
        )
        wb0.wait()
        wb1.wait()

    return gather_k


_sc_gathers = [_make_sc_gather(c * _BC) for c in range(_C)]

_BLK = 1024
_BLKS_PER_CHUNK = _BC // _BLK


def _silu_linear(x_ref, w_ref, b_ref, o_ref):
    x = x_ref[...]
    h = x * jax.nn.sigmoid(x)
    o_ref[...] = (
        lax.dot_general(
            h, w_ref[...], (((1,), (1,)), ((), ())),
            preferred_element_type=jnp.float32,
        )
        + b_ref[...]
    )


def _silu_linear_next(x_ref, w_ref, b_ref, _prev_ref, o_ref):
    _silu_linear(x_ref, w_ref, b_ref, o_ref)


def kernel(labels, table, W, b):
    labels = labels.astype(jnp.int32)
    b2 = b.reshape(1, EMB_DIM)

    gathered = [_sc_gathers[c](labels, table) for c in range(_C)]

    out_shape = jax.ShapeDtypeStruct((BATCH, EMB_DIM), jnp.float32)
    x_spec = pl.BlockSpec((_BLK, EMB_DIM), lambda i: (i, 0))
    w_spec = pl.BlockSpec((EMB_DIM, EMB_DIM), lambda i: (0, 0))
    b_spec = pl.BlockSpec((1, EMB_DIM), lambda i: (0, 0))

    out = pl.pallas_call(
        _silu_linear,
        grid=(_BLKS_PER_CHUNK,),
        in_specs=[x_spec, w_spec, b_spec],
        out_specs=pl.BlockSpec((_BLK, EMB_DIM), lambda i: (i, 0)),
        out_shape=out_shape,
    )(gathered[0], W, b2)

    for c in range(1, _C):
        off = c * _BLKS_PER_CHUNK
        out = pl.pallas_call(
            _silu_linear_next,
            grid=(_BLKS_PER_CHUNK,),
            in_specs=[
                x_spec,
                w_spec,
                b_spec,
                pl.BlockSpec(memory_space=pl.ANY),
            ],
            out_specs=pl.BlockSpec(
                (_BLK, EMB_DIM), lambda i, off=off: (i + off, 0)
            ),
            out_shape=out_shape,
            input_output_aliases={3: 0},
        )(gathered[c], W, b2, out)

    return out


# speedup vs baseline: 1.1009x; 1.1009x over previous
import functools

import jax
import jax.numpy as jnp
from jax import lax
from jax.experimental import pallas as pl
from jax.experimental.pallas import tpu as pltpu
from jax.experimental.pallas import tpu_sc as plsc

NUM_CLASSES = 100000
EMB_DIM = 128
BATCH = 16384

_NC = 2
_NS = 16
_NW = _NC * _NS
_BPW = BATCH // _NW
_SUB = _BPW // 2


def _make_sc_gather():
    mesh = plsc.VectorSubcoreMesh(core_axis_name="c", subcore_axis_name="s")

    @functools.partial(
        pl.kernel,
        mesh=mesh,
        out_type=jax.ShapeDtypeStruct((BATCH, EMB_DIM), jnp.float32),
        scratch_types=[
            pltpu.VMEM((_SUB,), jnp.int32),
            pltpu.VMEM((_SUB,), jnp.int32),
            pltpu.VMEM((_SUB, EMB_DIM), jnp.float32),
            pltpu.VMEM((_SUB, EMB_DIM), jnp.float32),
            pltpu.SemaphoreType.DMA,
            pltpu.SemaphoreType.DMA,
            pltpu.SemaphoreType.DMA,
            pltpu.SemaphoreType.DMA,
        ],
    )
    def gather_k(labels_hbm, table_hbm, out_hbm,
                 idx0_v, idx1_v, rows0_v, rows1_v, g0, g1, w0, w1):
        wid = lax.axis_index("s") * _NC + lax.axis_index("c")
        base = wid * _BPW
        pltpu.sync_copy(labels_hbm.at[pl.ds(base, _SUB)], idx0_v)
        cp0 = pltpu.async_copy(table_hbm.at[idx0_v], rows0_v, g0)
        pltpu.sync_copy(labels_hbm.at[pl.ds(base + _SUB, _SUB)], idx1_v)
        cp1 = pltpu.async_copy(table_hbm.at[idx1_v], rows1_v, g1)
        cp0.wait()
        wb0 = pltpu.async_copy(rows0_v, out_hbm.at[pl.ds(base, _SUB)], w0)
        cp1.wait()
        wb1 = pltpu.async_copy(
            rows1_v, out_hbm.at[pl.ds(base + _SUB, _SUB)], w1
        )
        wb0.wait()
        wb1.wait()

    return gather_k


_sc_gather = _make_sc_gather()

_BLK = 2048


def _silu_linear(x_ref, w_ref, b_ref, o_ref):
    x = x_ref[...]
    h = x * jax.nn.sigmoid(x)
    o_ref[...] = (
        lax.dot_general(
            h, w_ref[...], (((1,), (1,)), ((), ())),
            preferred_element_type=jnp.float32,
        )
        + b_ref[...]
    )


def kernel(labels, table, W, b):
    labels = labels.astype(jnp.int32)
    b2 = b.reshape(1, EMB_DIM)
    gathered = _sc_gather(labels, table)
    out = pl.pallas_call(
        _silu_linear,
        grid=(BATCH // _BLK,),
        in_specs=[
            pl.BlockSpec((_BLK, EMB_DIM), lambda i: (i, 0)),
            pl.BlockSpec((EMB_DIM, EMB_DIM), lambda i: (0, 0)),
            pl.BlockSpec((1, EMB_DIM), lambda i: (0, 0)),
        ],
        out_specs=pl.BlockSpec((_BLK, EMB_DIM), lambda i: (i, 0)),
        out_shape=jax.ShapeDtypeStruct((BATCH, EMB_DIM), jnp.float32),
    )(gathered, W, b2)
    return out


# speedup vs baseline: 1.2411x; 1.1273x over previous
import functools

import jax
import jax.numpy as jnp
from jax import lax
from jax.experimental import pallas as pl
from jax.experimental.pallas import tpu as pltpu
from jax.experimental.pallas import tpu_sc as plsc

NUM_CLASSES = 100000
EMB_DIM = 128
BATCH = 16384

_NC = 2
_NS = 16
_NW = _NC * _NS
_BPW = BATCH // _NW
_SUB = _BPW // 2


def _make_sc_gather():
    mesh = plsc.VectorSubcoreMesh(core_axis_name="c", subcore_axis_name="s")

    @functools.partial(
        pl.kernel,
        mesh=mesh,
        out_type=jax.ShapeDtypeStruct((BATCH, EMB_DIM), jnp.float32),
        scratch_types=[
            pltpu.VMEM((_SUB,), jnp.int32),
            pltpu.VMEM((_SUB,), jnp.int32),
            pltpu.VMEM((_SUB, EMB_DIM), jnp.float32),
            pltpu.VMEM((_SUB, EMB_DIM), jnp.float32),
            pltpu.SemaphoreType.DMA,
            pltpu.SemaphoreType.DMA,
            pltpu.SemaphoreType.DMA,
            pltpu.SemaphoreType.DMA,
        ],
    )
    def gather_k(labels_hbm, table_hbm, out_hbm,
                 idx0_v, idx1_v, rows0_v, rows1_v, g0, g1, w0, w1):
        wid = lax.axis_index("s") * _NC + lax.axis_index("c")
        base = wid * _BPW
        pltpu.sync_copy(labels_hbm.at[pl.ds(base, _SUB)], idx0_v)
        cp0 = pltpu.async_copy(table_hbm.at[idx0_v], rows0_v, g0)
        pltpu.sync_copy(labels_hbm.at[pl.ds(base + _SUB, _SUB)], idx1_v)
        cp1 = pltpu.async_copy(table_hbm.at[idx1_v], rows1_v, g1)
        cp0.wait()
        wb0 = pltpu.async_copy(rows0_v, out_hbm.at[pl.ds(base, _SUB)], w0)
        cp1.wait()
        wb1 = pltpu.async_copy(
            rows1_v, out_hbm.at[pl.ds(base + _SUB, _SUB)], w1
        )
        wb0.wait()
        wb1.wait()

    return gather_k


_sc_gather = _make_sc_gather()

_BLK = 8192


def _silu_linear(x_ref, w_ref, b_ref, o_ref):
    x = x_ref[...]
    h = x * jax.nn.sigmoid(x)
    o_ref[...] = (
        lax.dot_general(
            h, w_ref[...], (((1,), (1,)), ((), ())),
            preferred_element_type=jnp.float32,
        )
        + b_ref[...]
    )


def kernel(labels, table, W, b):
    labels = labels.astype(jnp.int32)
    b2 = b.reshape(1, EMB_DIM)
    gathered = _sc_gather(labels, table)
    out = pl.pallas_call(
        _silu_linear,
        grid=(BATCH // _BLK,),
        in_specs=[
            pl.BlockSpec((_BLK, EMB_DIM), lambda i: (i, 0)),
            pl.BlockSpec((EMB_DIM, EMB_DIM), lambda i: (0, 0)),
            pl.BlockSpec((1, EMB_DIM), lambda i: (0, 0)),
        ],
        out_specs=pl.BlockSpec((_BLK, EMB_DIM), lambda i: (i, 0)),
        out_shape=jax.ShapeDtypeStruct((BATCH, EMB_DIM), jnp.float32),
    )(gathered, W, b2)
    return out
